# baseline (speedup 1.0000x reference)
import functools

import jax
import jax.numpy as jnp
from jax import lax
from jax.experimental import pallas as pl
from jax.experimental.pallas import tpu as pltpu, tpu_sc as plsc

_N, _C = 100000, 240
_NT = _C // 8
_TPC = _NT // 2

_SRC = tuple(list(range(12, 20)) + list(range(0, 12)) + list(range(20, 30)))

_mesh = plsc.ScalarSubcoreMesh(axis_name="c")

_CHUNKS = tuple((i * 12544, 12544) for i in range(7)) + ((87808, 12160),)
_CB = 12544
_NBUF = 16
_DEPTH = 8


@functools.partial(
    pl.kernel,
    out_type=jax.ShapeDtypeStruct((_C, _N), jnp.float32),
    mesh=_mesh,
    scratch_types=[pltpu.SemaphoreType.DMA for _ in range(_NBUF)],
)
def _sc_permute_t(xt_hbm, ot_hbm, *sems):
    core = lax.axis_index("c")
    d0 = core * _TPC

    def make_copy(t):
        s_lo = 8 * _SRC[t]
        s_hi = 8 * _SRC[_TPC + t]
        s = lax.select(core == 0, jnp.int32(s_lo), jnp.int32(s_hi))
        s = pl.multiple_of(s, 8)
        d = (d0 + t) * 8
        return pltpu.make_async_copy(
            xt_hbm.at[pl.ds(s, 8)],
            ot_hbm.at[pl.ds(d, 8)],
            sems[t % _NBUF],
        )

    for t in range(_TPC):
        if t >= _NBUF:
            make_copy(t - _NBUF).wait()
        make_copy(t).start()
    for t in range(max(0, _TPC - _NBUF), _TPC):
        make_copy(t).wait()


_TAIL0 = (_N // 128) * 128
_TAILN = _N - _TAIL0


def _tail_body(x_ref, o_ref):
    x = x_ref[...]
    o_ref[:, 0:64] = x[:, 96:160]
    o_ref[:, 64:160] = x[:, 0:96]
    o_ref[:, 160:240] = x[:, 160:240]


def _tail_permute(xtail):
    return pl.pallas_call(
        _tail_body,
        out_shape=jax.ShapeDtypeStruct((_TAILN, _C), jnp.float32),
    )(xtail)


def kernel(x):
    yt = _sc_permute_t(x.T)
    y = yt.T
    ytail = _tail_permute(jax.lax.dynamic_slice(x, (_TAIL0, 0), (_TAILN, _C)))
    return jax.lax.dynamic_update_slice(y, ytail, (_TAIL0, 0))


# speedup vs baseline: 33.8198x; 33.8198x over previous
import functools

import jax
import jax.numpy as jnp
from jax import lax
from jax.experimental import pallas as pl
from jax.experimental.pallas import tpu as pltpu, tpu_sc as plsc

_N, _C = 100000, 240
_NT = _C // 8
_TPC = _NT // 2

_SRC = tuple(list(range(12, 20)) + list(range(0, 12)) + list(range(20, 30)))

_mesh = plsc.VectorSubcoreMesh(core_axis_name="c", subcore_axis_name="s")
_NC, _NS = 2, 16
_NW = _NC * _NS
_LW_BIG = 25 * 128
_LW_SMALL = 24 * 128
_NBUF = 4


@functools.partial(
    pl.kernel,
    out_type=jax.ShapeDtypeStruct((_C, _N), jnp.float32),
    mesh=_mesh,
    scratch_types=(
        [pltpu.VMEM((8, _LW_BIG), jnp.float32) for _ in range(_NBUF)]
        + [pltpu.SemaphoreType.DMA for _ in range(2 * _NBUF)]
    ),
)
def _sc_permute_t(xt_hbm, ot_hbm, *sc):
    bufs = sc[:_NBUF]
    isems = sc[_NBUF:2 * _NBUF]
    osems = sc[2 * _NBUF:]
    c = lax.axis_index("c")
    sidx = lax.axis_index("s")
    w = sidx * _NC + c
    lane0 = (24 * w + jnp.minimum(w, 13)) * 128
    lane0 = pl.multiple_of(lane0, 128)

    def run(width):
        def mk_in(t):
            return pltpu.make_async_copy(
                xt_hbm.at[pl.ds(8 * _SRC[t], 8), pl.ds(lane0, width)],
                bufs[t % _NBUF].at[:, pl.ds(0, width)],
                isems[t % _NBUF],
            )

        def mk_out(t):
            return pltpu.make_async_copy(
                bufs[t % _NBUF].at[:, pl.ds(0, width)],
                ot_hbm.at[pl.ds(8 * t, 8), pl.ds(lane0, width)],
                osems[t % _NBUF],
            )

        mk_in(0).start()
        mk_in(1).start()
        for t in range(_NT):
            mk_in(t).wait()
            mk_out(t).start()
            nt = t + 2
            if nt < _NT:
                if nt >= _NBUF:
                    mk_out(nt - _NBUF).wait()
                mk_in(nt).start()
        for t in range(_NT - _NBUF, _NT):
            mk_out(t).wait()

    @pl.when(w < 13)
    def _():
        run(_LW_BIG)

    @pl.when(w >= 13)
    def _():
        run(_LW_SMALL)


_TAIL0 = (_N // 128) * 128
_TAILN = _N - _TAIL0


def _tail_body(x_ref, o_ref):
    x = x_ref[...]
    o_ref[:, 0:64] = x[:, 96:160]
    o_ref[:, 64:160] = x[:, 0:96]
    o_ref[:, 160:240] = x[:, 160:240]


def _tail_permute(xtail):
    return pl.pallas_call(
        _tail_body,
        out_shape=jax.ShapeDtypeStruct((_TAILN, _C), jnp.float32),
    )(xtail)


def kernel(x):
    yt = _sc_permute_t(x.T)
    y = yt.T
    ytail = _tail_permute(jax.lax.dynamic_slice(x, (_TAIL0, 0), (_TAILN, _C)))
    return jax.lax.dynamic_update_slice(y, ytail, (_TAIL0, 0))


# speedup vs baseline: 36.1367x; 1.0685x over previous
import functools

import jax
import jax.numpy as jnp
from jax import lax
from jax.experimental import pallas as pl
from jax.experimental.pallas import tpu as pltpu, tpu_sc as plsc

_N, _C = 100000, 240
_NT = _C // 8
_TPC = _NT // 2

_SRC = tuple(list(range(12, 20)) + list(range(0, 12)) + list(range(20, 30)))

_mesh = plsc.ScalarSubcoreMesh(axis_name="c")

_CHUNKS = tuple((i * 12544, 12544) for i in range(7)) + ((87808, 12160),)
_CB = 12544
_NBUF = 16
_DEPTH = 12


@functools.partial(
    pl.kernel,
    out_type=jax.ShapeDtypeStruct((_C, _N), jnp.float32),
    mesh=_mesh,
    scratch_types=(
        [pltpu.VMEM_SHARED((8, _CB), jnp.float32) for _ in range(_NBUF)]
        + [pltpu.SemaphoreType.DMA for _ in range(2 * _NBUF)]
    ),
)
def _sc_permute_t(xt_hbm, ot_hbm, *sc):
    bufs = sc[:_NBUF]
    isems = sc[_NBUF:2 * _NBUF]
    osems = sc[2 * _NBUF:]
    core = lax.axis_index("c")
    d0 = core * _TPC

    items = [(t, c) for t in range(_TPC) for c in range(len(_CHUNKS))]

    def make_in(i):
        t, c = items[i]
        lo, sz = _CHUNKS[c]
        s_lo = 8 * _SRC[t]
        s_hi = 8 * _SRC[_TPC + t]
        s = lax.select(core == 0, jnp.int32(s_lo), jnp.int32(s_hi))
        s = pl.multiple_of(s, 8)
        return pltpu.make_async_copy(
            xt_hbm.at[pl.ds(s, 8), pl.ds(lo, sz)],
            bufs[i % _NBUF].at[:, pl.ds(0, sz)],
            isems[i % _NBUF],
        )

    def make_out(i):
        t, c = items[i]
        lo, sz = _CHUNKS[c]
        d = (d0 + t) * 8
        return pltpu.make_async_copy(
            bufs[i % _NBUF].at[:, pl.ds(0, sz)],
            ot_hbm.at[pl.ds(d, 8), pl.ds(lo, sz)],
            osems[i % _NBUF],
        )

    n = len(items)
    for i in range(min(_DEPTH, n)):
        make_in(i).start()
    for i in range(n):
        make_in(i).wait()
        make_out(i).start()
        ni = i + _DEPTH
        if ni < n:
            if ni >= _NBUF:
                make_out(ni - _NBUF).wait()
            make_in(ni).start()
    for i in range(max(0, n - _NBUF), n):
        make_out(i).wait()


_TAIL0 = (_N // 128) * 128
_TAILN = _N - _TAIL0


def _tail_body(x_ref, o_ref):
    x = x_ref[...]
    o_ref[:, 0:64] = x[:, 96:160]
    o_ref[:, 64:160] = x[:, 0:96]
    o_ref[:, 160:240] = x[:, 160:240]


def _tail_permute(xtail):
    return pl.pallas_call(
        _tail_body,
        out_shape=jax.ShapeDtypeStruct((_TAILN, _C), jnp.float32),
    )(xtail)


def kernel(x):
    yt = _sc_permute_t(x.T)
    y = yt.T
    ytail = _tail_permute(jax.lax.dynamic_slice(x, (_TAIL0, 0), (_TAILN, _C)))
    return jax.lax.dynamic_update_slice(y, ytail, (_TAIL0, 0))


# speedup vs baseline: 36.6516x; 1.0142x over previous
import functools

import jax
import jax.numpy as jnp
from jax import lax
from jax.experimental import pallas as pl
from jax.experimental.pallas import tpu as pltpu, tpu_sc as plsc

_N, _C = 100000, 240
_NT = _C // 8
_TPC = _NT // 2

_SRC = tuple(list(range(12, 20)) + list(range(0, 12)) + list(range(20, 30)))

_mesh = plsc.ScalarSubcoreMesh(axis_name="c")

_CHUNKS = tuple((i * 12544, 12544) for i in range(7)) + ((87808, 12160),)
_CB = 12544
_NBUF = 16
_DEPTH = 8


@functools.partial(
    pl.kernel,
    out_type=jax.ShapeDtypeStruct((_C, _N), jnp.float32),
    mesh=_mesh,
    scratch_types=(
        [pltpu.VMEM_SHARED((8, _CB), jnp.float32) for _ in range(_NBUF)]
        + [pltpu.SemaphoreType.DMA for _ in range(2 * _NBUF)]
    ),
)
def _sc_permute_t(xt_hbm, ot_hbm, *sc):
    bufs = sc[:_NBUF]
    isems = sc[_NBUF:2 * _NBUF]
    osems = sc[2 * _NBUF:]
    core = lax.axis_index("c")
    d0 = core * _TPC

    items = [(t, c) for t in range(_TPC) for c in range(len(_CHUNKS))]

    def make_in(i):
        t, c = items[i]
        lo, sz = _CHUNKS[c]
        s_lo = 8 * _SRC[t]
        s_hi = 8 * _SRC[_TPC + t]
        s = lax.select(core == 0, jnp.int32(s_lo), jnp.int32(s_hi))
        s = pl.multiple_of(s, 8)
        return pltpu.make_async_copy(
            xt_hbm.at[pl.ds(s, 8), pl.ds(lo, sz)],
            bufs[i % _NBUF].at[:, pl.ds(0, sz)],
            isems[i % _NBUF],
        )

    def make_out(i):
        t, c = items[i]
        lo, sz = _CHUNKS[c]
        d = (d0 + t) * 8
        return pltpu.make_async_copy(
            bufs[i % _NBUF].at[:, pl.ds(0, sz)],
            ot_hbm.at[pl.ds(d, 8), pl.ds(lo, sz)],
            osems[i % _NBUF],
        )

    n = len(items)
    for i in range(min(_DEPTH, n)):
        make_in(i).start()
    for i in range(n):
        make_in(i).wait()
        make_out(i).start()
        ni = i + _DEPTH
        if ni < n:
            if ni >= _NBUF:
                make_out(ni - _NBUF).wait()
            make_in(ni).start()
    for i in range(max(0, n - _NBUF), n):
        make_out(i).wait()


_TAIL0 = (_N // 128) * 128
_TAILN = _N - _TAIL0


def _tail_body(x_ref, o_ref):
    x = x_ref[...]
    o_ref[:, 0:64] = x[:, 96:160]
    o_ref[:, 64:160] = x[:, 0:96]
    o_ref[:, 160:240] = x[:, 160:240]


def _tail_permute(xtail):
    return pl.pallas_call(
        _tail_body,
        out_shape=jax.ShapeDtypeStruct((_TAILN, _C), jnp.float32),
    )(xtail)


def kernel(x):
    yt = _sc_permute_t(x.T)
    y = yt.T
    ytail = _tail_permute(jax.lax.dynamic_slice(x, (_TAIL0, 0), (_TAILN, _C)))
    return jax.lax.dynamic_update_slice(y, ytail, (_TAIL0, 0))
